# baseline (speedup 1.0000x reference)
import jax
import jax.numpy as jnp
from jax import lax
from jax.experimental import pallas as pl
from jax.experimental.pallas import tpu as pltpu
from jax.experimental.pallas import tpu_sc as plsc

N = 4096
NC = 2
NS = 16
NW = NC * NS
RW = N // NW
GR = 8


def _sc_body(q_hbm, at_hbm, x_hbm, y_hbm, out_hbm, xv, yv, qbuf, abuf,
             accbuf, sem):
    wid = lax.axis_index("s") * NC + lax.axis_index("c")
    base = wid * RW
    pltpu.sync_copy(x_hbm, xv)
    pltpu.sync_copy(y_hbm, yv)

    def row_chunks(buf, vec, rr):
        def chunk(j, acc):
            sl = pl.ds(j * 16, 16)
            return acc + buf[rr, sl] * vec[sl]
        return lax.fori_loop(0, N // 16, chunk,
                             jnp.zeros((16,), jnp.float32), unroll=8)

    def group(g, carry):
        rb = base + g * GR
        cq = pltpu.async_copy(q_hbm.at[pl.ds(rb, GR)], qbuf, sem)
        ca = pltpu.async_copy(at_hbm.at[pl.ds(rb, GR)], abuf, sem)
        cq.wait()
        ca.wait()

        def row(rr, c2):
            accbuf[g * GR + rr, :] = (row_chunks(qbuf, xv, rr)
                                      + row_chunks(abuf, yv, rr))
            return c2

        return lax.fori_loop(0, GR, row, carry)

    lax.fori_loop(0, RW // GR, group, jnp.int32(0))
    pltpu.sync_copy(accbuf, out_hbm.at[pl.ds(base, RW)])


def _combine_body(p_ref, c_ref, out_ref):
    pg = jnp.sum(p_ref[...], axis=1, keepdims=True) + c_ref[...]
    out_ref[0, 0] = jnp.max(jnp.abs(pg)) / (1.0 + jnp.max(jnp.abs(c_ref[...])))


def kernel(Q, AT, b, c, x, y, Iy, il, iu, l, u):
    xf = x.reshape(N)
    yf = y.reshape(N)
    c2 = c.reshape(N, 1)
    mesh = plsc.VectorSubcoreMesh(core_axis_name="c", subcore_axis_name="s")
    partials = pl.kernel(
        _sc_body,
        out_type=jax.ShapeDtypeStruct((N, 16), jnp.float32),
        mesh=mesh,
        scratch_types=[
            pltpu.VMEM((N,), jnp.float32),
            pltpu.VMEM((N,), jnp.float32),
            pltpu.VMEM((GR, N), jnp.float32),
            pltpu.VMEM((GR, N), jnp.float32),
            pltpu.VMEM((RW, 16), jnp.float32),
            pltpu.SemaphoreType.DMA,
        ],
    )(Q, AT, xf, yf)
    out = pl.pallas_call(
        _combine_body,
        out_specs=pl.BlockSpec(memory_space=pltpu.SMEM),
        out_shape=jax.ShapeDtypeStruct((1, 1), jnp.float32),
    )(partials, c2)
    return out[0, 0]


# speedup vs baseline: 2.1900x; 2.1900x over previous
import jax
import jax.numpy as jnp
from jax import lax
from jax.experimental import pallas as pl
from jax.experimental.pallas import tpu as pltpu
from jax.experimental.pallas import tpu_sc as plsc

N = 4096
RS = 1024
NC = 2
NS = 16
NW = NC * NS
RW = RS // NW
GR = 8
BM = 256


def _sc_body(q_hbm, at_hbm, x_hbm, y_hbm, out_hbm, xv, yv, qbuf, abuf,
             accbuf, sem):
    wid = lax.axis_index("s") * NC + lax.axis_index("c")
    base = wid * RW
    pltpu.sync_copy(x_hbm, xv)
    pltpu.sync_copy(y_hbm, yv)

    def row_chunks(buf, vec, rr):
        def chunk(j, acc):
            sl = pl.ds(j * 16, 16)
            return acc + buf[rr, sl] * vec[sl]
        return lax.fori_loop(0, N // 16, chunk,
                             jnp.zeros((16,), jnp.float32), unroll=8)

    def group(g, carry):
        rb = base + g * GR
        cq = pltpu.async_copy(q_hbm.at[pl.ds(rb, GR)], qbuf, sem)
        ca = pltpu.async_copy(at_hbm.at[pl.ds(rb, GR)], abuf, sem)
        cq.wait()
        ca.wait()

        def row(rr, c2):
            accbuf[g * GR + rr, :] = (row_chunks(qbuf, xv, rr)
                                      + row_chunks(abuf, yv, rr))
            return c2

        return lax.fori_loop(0, GR, row, carry)

    lax.fori_loop(0, RW // GR, group, jnp.int32(0))
    pltpu.sync_copy(accbuf, out_hbm.at[pl.ds(base, RW)])


def _tc_body(q_ref, at_ref, xt_ref, yt_ref, c_ref, cfull_ref, gmax_ref,
             mc_ref):
    i = pl.program_id(0)
    qx = jnp.sum(q_ref[...] * xt_ref[...], axis=1, keepdims=True)
    aty = jnp.sum(at_ref[...] * yt_ref[...], axis=1, keepdims=True)
    pg = qx + aty + c_ref[...]
    m = jnp.max(jnp.abs(pg))

    @pl.when(i == 0)
    def _init():
        gmax_ref[0, 0] = m
        mc_ref[0, 0] = jnp.max(jnp.abs(cfull_ref[...]))

    @pl.when(i > 0)
    def _acc():
        gmax_ref[0, 0] = jnp.maximum(gmax_ref[0, 0], m)


def _combine_body(p_ref, ch_ref, gmax_ref, mc_ref, out_ref):
    pg = jnp.sum(p_ref[...], axis=1, keepdims=True) + ch_ref[...]
    m = jnp.maximum(jnp.max(jnp.abs(pg)), gmax_ref[0, 0])
    out_ref[0, 0] = m / (1.0 + mc_ref[0, 0])


def kernel(Q, AT, b, c, x, y, Iy, il, iu, l, u):
    xf = x.reshape(N)
    yf = y.reshape(N)
    c2 = c.reshape(N, 1)
    mesh = plsc.VectorSubcoreMesh(core_axis_name="c", subcore_axis_name="s")
    partials = pl.kernel(
        _sc_body,
        out_type=jax.ShapeDtypeStruct((RS, 16), jnp.float32),
        mesh=mesh,
        scratch_types=[
            pltpu.VMEM((N,), jnp.float32),
            pltpu.VMEM((N,), jnp.float32),
            pltpu.VMEM((GR, N), jnp.float32),
            pltpu.VMEM((GR, N), jnp.float32),
            pltpu.VMEM((RW, 16), jnp.float32),
            pltpu.SemaphoreType.DMA,
        ],
    )(Q, AT, xf, yf)
    grid = (N - RS) // BM
    gmax, mc = pl.pallas_call(
        _tc_body,
        grid=(grid,),
        in_specs=[
            pl.BlockSpec((BM, N), lambda i: (i + RS // BM, 0)),
            pl.BlockSpec((BM, N), lambda i: (i + RS // BM, 0)),
            pl.BlockSpec((1, N), lambda i: (0, 0)),
            pl.BlockSpec((1, N), lambda i: (0, 0)),
            pl.BlockSpec((BM, 1), lambda i: (i + RS // BM, 0)),
            pl.BlockSpec((N, 1), lambda i: (0, 0)),
        ],
        out_specs=[
            pl.BlockSpec(memory_space=pltpu.SMEM),
            pl.BlockSpec(memory_space=pltpu.SMEM),
        ],
        out_shape=[
            jax.ShapeDtypeStruct((1, 1), jnp.float32),
            jax.ShapeDtypeStruct((1, 1), jnp.float32),
        ],
    )(Q, AT, xf.reshape(1, N), yf.reshape(1, N), c2, c2)
    out = pl.pallas_call(
        _combine_body,
        in_specs=[
            pl.BlockSpec((RS, 16), lambda: (0, 0)),
            pl.BlockSpec((RS, 1), lambda: (0, 0)),
            pl.BlockSpec(memory_space=pltpu.SMEM),
            pl.BlockSpec(memory_space=pltpu.SMEM),
        ],
        out_specs=pl.BlockSpec(memory_space=pltpu.SMEM),
        out_shape=jax.ShapeDtypeStruct((1, 1), jnp.float32),
    )(partials, c2[:RS], gmax, mc)
    return out[0, 0]


# speedup vs baseline: 3.3962x; 1.5508x over previous
import jax
import jax.numpy as jnp
from jax.experimental import pallas as pl
from jax.experimental.pallas import tpu as pltpu

N = 4096
BM = 256


def _body(q_ref, at_ref, xt_ref, yt_ref, c_ref, out_ref, gmax_ref, cmax_ref):
    i = pl.program_id(0)
    qx = jnp.sum(q_ref[...] * xt_ref[...], axis=1, keepdims=True)
    aty = jnp.sum(at_ref[...] * yt_ref[...], axis=1, keepdims=True)
    pg = qx + aty + c_ref[...]
    m = jnp.max(jnp.abs(pg))
    mc = jnp.max(jnp.abs(c_ref[...]))

    @pl.when(i == 0)
    def _init():
        gmax_ref[0, 0] = m
        cmax_ref[0, 0] = mc

    @pl.when(i > 0)
    def _acc():
        gmax_ref[0, 0] = jnp.maximum(gmax_ref[0, 0], m)
        cmax_ref[0, 0] = jnp.maximum(cmax_ref[0, 0], mc)

    @pl.when(i == pl.num_programs(0) - 1)
    def _fin():
        out_ref[0, 0] = gmax_ref[0, 0] / (1.0 + cmax_ref[0, 0])


def kernel(Q, AT, b, c, x, y, Iy, il, iu, l, u):
    xt = x.reshape(1, N)
    yt = y.reshape(1, N)
    c2 = c.reshape(N, 1)
    grid = N // BM
    out = pl.pallas_call(
        _body,
        grid=(grid,),
        in_specs=[
            pl.BlockSpec((BM, N), lambda i: (i, 0)),
            pl.BlockSpec((BM, N), lambda i: (i, 0)),
            pl.BlockSpec((1, N), lambda i: (0, 0)),
            pl.BlockSpec((1, N), lambda i: (0, 0)),
            pl.BlockSpec((BM, 1), lambda i: (i, 0)),
        ],
        out_specs=pl.BlockSpec(memory_space=pltpu.SMEM),
        out_shape=jax.ShapeDtypeStruct((1, 1), jnp.float32),
        scratch_shapes=[
            pltpu.SMEM((1, 1), jnp.float32),
            pltpu.SMEM((1, 1), jnp.float32),
        ],
    )(Q, AT, xt, yt, c2)
    return out[0, 0]


# speedup vs baseline: 3.7542x; 1.1054x over previous
import jax
import jax.numpy as jnp
from jax.experimental import pallas as pl
from jax.experimental.pallas import tpu as pltpu

N = 4096
BM = 256


def _body(q_ref, at_ref, xt_ref, yt_ref, c_ref, out_ref, gmax_ref, cmax_ref):
    i = pl.program_id(0)
    qx = jnp.sum(q_ref[...] * xt_ref[...], axis=1)
    aty = jnp.sum(at_ref[...] * yt_ref[...], axis=1)
    ct = c_ref[0, pl.ds(i * BM, BM)]
    pg = qx + aty + ct
    m = jnp.max(jnp.abs(pg))

    @pl.when(i == 0)
    def _init():
        gmax_ref[0, 0] = m
        cmax_ref[0, 0] = jnp.max(jnp.abs(c_ref[...]))

    @pl.when(i > 0)
    def _acc():
        gmax_ref[0, 0] = jnp.maximum(gmax_ref[0, 0], m)

    @pl.when(i == pl.num_programs(0) - 1)
    def _fin():
        out_ref[0, 0] = gmax_ref[0, 0] / (1.0 + cmax_ref[0, 0])


def kernel(Q, AT, b, c, x, y, Iy, il, iu, l, u):
    xt = x.reshape(1, N)
    yt = y.reshape(1, N)
    crow = c.reshape(1, N)
    grid = N // BM
    out = pl.pallas_call(
        _body,
        grid=(grid,),
        in_specs=[
            pl.BlockSpec((BM, N), lambda i: (i, 0)),
            pl.BlockSpec((BM, N), lambda i: (i, 0)),
            pl.BlockSpec((1, N), lambda i: (0, 0)),
            pl.BlockSpec((1, N), lambda i: (0, 0)),
            pl.BlockSpec((1, N), lambda i: (0, 0)),
        ],
        out_specs=pl.BlockSpec(memory_space=pltpu.SMEM),
        out_shape=jax.ShapeDtypeStruct((1, 1), jnp.float32),
        scratch_shapes=[
            pltpu.SMEM((1, 1), jnp.float32),
            pltpu.SMEM((1, 1), jnp.float32),
        ],
    )(Q, AT, xt, yt, crow)
    return out[0, 0]
